# Initial kernel scaffold; baseline (speedup 1.0000x reference)
import jax
import jax.numpy as jnp
from jax.experimental import pallas as pl


def kernel(x, weight, bias):
    raise NotImplementedError("write your pallas kernel here")


# speedup vs baseline: 2.3295x; 2.3295x over previous
import functools

import jax
import jax.numpy as jnp
from jax.experimental import pallas as pl
from jax.experimental.pallas import tpu as pltpu

_B, _S, _H = 4, 4096, 4096
_E = 64
_K = 8
_G = 64
_T = 512


def _gate_kernel(x_ref, wt_ref, b_ref, idx_ref, w_ref, aux_ref,
                 cnt_acc, ssum_acc):
    pid = pl.program_id(0)
    nsteps = pl.num_programs(0)

    @pl.when(pid == 0)
    def _init():
        cnt_acc[...] = jnp.zeros_like(cnt_acc)
        ssum_acc[...] = jnp.zeros_like(ssum_acc)

    x = x_ref[...]
    wt = wt_ref[...]
    logits = jnp.dot(x, wt, preferred_element_type=jnp.float32) + b_ref[...]

    mx = jnp.max(logits, axis=-1, keepdims=True)
    ex = jnp.exp(logits - mx)
    scores = ex / jnp.sum(ex, axis=-1, keepdims=True)

    iota = jax.lax.broadcasted_iota(jnp.int32, scores.shape, 1)
    vals = scores
    top_vals = []
    top_idx = []
    for _ in range(_K):
        m = jnp.max(vals, axis=-1, keepdims=True)
        sel = vals == m
        idx = jnp.min(jnp.where(sel, iota, _E), axis=-1, keepdims=True)
        top_vals.append(m)
        top_idx.append(idx)
        vals = jnp.where(iota == idx, -1.0, vals)

    tv = jnp.concatenate(top_vals, axis=1)
    ti = jnp.concatenate(top_idx, axis=1)
    denom = jnp.sum(tv, axis=-1, keepdims=True) + 1e-20
    idx_ref[...] = ti
    w_ref[...] = tv / denom

    blocks_per_batch = _S // _T
    b = pid // blocks_per_batch
    selected = (vals < -0.5).astype(jnp.float32)
    cnt = jnp.sum(selected, axis=0, keepdims=True)
    ssum = jnp.sum(scores, axis=0, keepdims=True)
    rows = jax.lax.broadcasted_iota(jnp.int32, (_B, _E), 0)
    hit = (rows == b).astype(jnp.float32)
    cnt_acc[...] += hit * cnt
    ssum_acc[...] += hit * ssum

    @pl.when(pid == nsteps - 1)
    def _finish():
        scale = _G / (_S * _K * _S * _B)
        aux_ref[...] = (jnp.sum(cnt_acc[...] * ssum_acc[...]) * scale).reshape(1, 1)


@functools.partial(jax.jit)
def _run(x, weight, bias):
    hidden = x.reshape(-1, _H)
    wt = weight.T
    b2 = bias.reshape(1, _E)
    n = hidden.shape[0]
    grid = (n // _T,)
    out = pl.pallas_call(
        _gate_kernel,
        grid=grid,
        in_specs=[
            pl.BlockSpec((_T, _H), lambda i: (i, 0)),
            pl.BlockSpec((_H, _E), lambda i: (0, 0)),
            pl.BlockSpec((1, _E), lambda i: (0, 0)),
        ],
        out_specs=[
            pl.BlockSpec((_T, _K), lambda i: (i, 0)),
            pl.BlockSpec((_T, _K), lambda i: (i, 0)),
            pl.BlockSpec((1, 1), lambda i: (0, 0)),
        ],
        out_shape=[
            jax.ShapeDtypeStruct((n, _K), jnp.int32),
            jax.ShapeDtypeStruct((n, _K), jnp.float32),
            jax.ShapeDtypeStruct((1, 1), jnp.float32),
        ],
        scratch_shapes=[
            pltpu.VMEM((_B, _E), jnp.float32),
            pltpu.VMEM((_B, _E), jnp.float32),
        ],
    )(hidden, wt, b2)
    topk_idx, topk_weight, aux = out
    return topk_idx, topk_weight, aux[0, 0]


def kernel(x, weight, bias):
    return _run(x, weight, bias)
